# baseline (speedup 1.0000x reference)
import functools

import jax
import jax.numpy as jnp
from jax import lax
from jax.experimental import pallas as pl
from jax.experimental.pallas import tpu as pltpu
from jax.experimental.pallas import tpu_sc as plsc

B = 16384
EMB = 64
TDIM = 10

_info = plsc.get_sparse_core_info()
_NC, _NS = _info.num_cores, _info.num_subcores
_NW = _NC * _NS
_BPW = B // _NW


@functools.partial(
    pl.kernel,
    mesh=plsc.VectorSubcoreMesh(core_axis_name="c", subcore_axis_name="s"),
    compiler_params=pltpu.CompilerParams(use_tc_tiling_on_sc=False),
    out_type=[
        jax.ShapeDtypeStruct((B, EMB), jnp.float32),
        jax.ShapeDtypeStruct((B, EMB), jnp.float32),
        jax.ShapeDtypeStruct((B, 1), jnp.float32),
        jax.ShapeDtypeStruct((B, 1), jnp.float32),
    ],
    scratch_types=[
        pltpu.VMEM((_BPW,), jnp.int32),
        pltpu.VMEM((_BPW,), jnp.int32),
        pltpu.VMEM((_BPW, EMB), jnp.float32),
        pltpu.VMEM((_BPW, EMB), jnp.float32),
        pltpu.VMEM((_BPW, 1), jnp.float32),
        pltpu.VMEM((_BPW, 1), jnp.float32),
        pltpu.SemaphoreType.DMA,
        pltpu.SemaphoreType.DMA,
        pltpu.SemaphoreType.DMA,
        pltpu.SemaphoreType.DMA,
    ],
)
def _sc_gather(rest_table, menu_table, rest_bias, menu_bias, ridx, midx,
               rest_out, menu_out, rb_out, mb_out,
               ridx_v, midx_v, rrows_v, mrows_v, rb_v, mb_v,
               sem_r, sem_m, sem_rb, sem_mb):
    wid = lax.axis_index("s") * _NC + lax.axis_index("c")
    base = wid * _BPW
    pltpu.sync_copy(ridx.at[pl.ds(base, _BPW)], ridx_v)
    pltpu.sync_copy(midx.at[pl.ds(base, _BPW)], midx_v)
    c_r = pltpu.async_copy(rest_table.at[ridx_v], rrows_v, sem_r)
    c_m = pltpu.async_copy(menu_table.at[midx_v], mrows_v, sem_m)
    c_rb = pltpu.async_copy(rest_bias.at[ridx_v], rb_v, sem_rb)
    c_mb = pltpu.async_copy(menu_bias.at[midx_v], mb_v, sem_mb)
    c_r.wait()
    c_m.wait()
    c_rb.wait()
    c_mb.wait()
    pltpu.sync_copy(rrows_v, rest_out.at[pl.ds(base, _BPW)])
    pltpu.sync_copy(mrows_v, menu_out.at[pl.ds(base, _BPW)])
    pltpu.sync_copy(rb_v, rb_out.at[pl.ds(base, _BPW)])
    pltpu.sync_copy(mb_v, mb_out.at[pl.ds(base, _BPW)])


def _dense_body(rest_ref, menu_ref, temp_ref, rb_ref, mb_ref, gb_ref,
                tW1_ref, tb1_ref, tW2_ref, tb2_ref, tW3_ref, tb3_ref,
                iW1r_ref, iW1m_ref, iW1t_ref, ib1_ref,
                iW2_ref, ib2_ref, iW3_ref, ib3_ref, out_ref):
    f32 = jnp.float32
    rest = rest_ref[...]
    menu = menu_ref[...]
    temp = temp_ref[...]
    mf = jnp.sum(rest * menu, axis=1, keepdims=True)
    h = jnp.maximum(
        jnp.dot(temp, tW1_ref[...], preferred_element_type=f32) + tb1_ref[...], 0.0)
    h = jnp.maximum(
        jnp.dot(h, tW2_ref[...], preferred_element_type=f32) + tb2_ref[...], 0.0)
    t_score = jnp.dot(h, tW3_ref[...], preferred_element_type=f32) + tb3_ref[...]
    g = (jnp.dot(rest, iW1r_ref[...], preferred_element_type=f32)
         + jnp.dot(menu, iW1m_ref[...], preferred_element_type=f32)
         + jnp.dot(temp, iW1t_ref[...], preferred_element_type=f32)
         + ib1_ref[...])
    g = jnp.maximum(g, 0.0)
    g = jnp.maximum(
        jnp.dot(g, iW2_ref[...], preferred_element_type=f32) + ib2_ref[...], 0.0)
    i_score = jnp.dot(g, iW3_ref[...], preferred_element_type=f32) + ib3_ref[...]
    pred = gb_ref[...] + rb_ref[...] + mb_ref[...] + mf + t_score + i_score
    out_ref[...] = jax.nn.sigmoid(pred[:, 0])


def kernel(restaurant_idx, menu_idx, temporal_features, rest_table, menu_table,
           rest_bias_table, menu_bias_table, global_bias, tW1, tb1, tW2, tb2,
           tW3, tb3, iW1, ib1, iW2, ib2, iW3, ib3):
    ridx = restaurant_idx.astype(jnp.int32)
    midx = menu_idx.astype(jnp.int32)
    rest_emb, menu_emb, rest_b, menu_b = _sc_gather(
        rest_table, menu_table, rest_bias_table, menu_bias_table, ridx, midx)

    BS = 2048
    grid = (B // BS,)
    full = lambda shape: pl.BlockSpec(shape, lambda i: (0, 0))
    out = pl.pallas_call(
        _dense_body,
        grid=grid,
        in_specs=[
            pl.BlockSpec((BS, EMB), lambda i: (i, 0)),
            pl.BlockSpec((BS, EMB), lambda i: (i, 0)),
            pl.BlockSpec((BS, TDIM), lambda i: (i, 0)),
            pl.BlockSpec((BS, 1), lambda i: (i, 0)),
            pl.BlockSpec((BS, 1), lambda i: (i, 0)),
            full((1, 1)),
            full((TDIM, 32)), full((1, 32)),
            full((32, 16)), full((1, 16)),
            full((16, 1)), full((1, 1)),
            full((EMB, 128)), full((EMB, 128)), full((TDIM, 128)), full((1, 128)),
            full((128, 64)), full((1, 64)),
            full((64, 1)), full((1, 1)),
        ],
        out_specs=pl.BlockSpec((BS,), lambda i: (i,)),
        out_shape=jax.ShapeDtypeStruct((B,), jnp.float32),
    )(
        rest_emb, menu_emb, temporal_features, rest_b, menu_b,
        global_bias.reshape(1, 1),
        tW1, tb1.reshape(1, 32), tW2, tb2.reshape(1, 16),
        tW3, tb3.reshape(1, 1),
        iW1[:EMB], iW1[EMB:2 * EMB], iW1[2 * EMB:], ib1.reshape(1, 128),
        iW2, ib2.reshape(1, 64), iW3, ib3.reshape(1, 1),
    )
    return out


# speedup vs baseline: 3.3578x; 3.3578x over previous
import functools

import jax
import jax.numpy as jnp
from jax import lax
from jax.experimental import pallas as pl
from jax.experimental.pallas import tpu as pltpu
from jax.experimental.pallas import tpu_sc as plsc

B = 16384
EMB = 64
TDIM = 10

_info = plsc.get_sparse_core_info()
_NC, _NS = _info.num_cores, _info.num_subcores
_NW = _NC * _NS
_BPW = B // _NW
_FPW = _BPW * EMB


@functools.partial(
    pl.kernel,
    mesh=plsc.VectorSubcoreMesh(core_axis_name="c", subcore_axis_name="s"),
    compiler_params=pltpu.CompilerParams(needs_layout_passes=False),
    out_type=[
        jax.ShapeDtypeStruct((B, EMB), jnp.float32),
        jax.ShapeDtypeStruct((B, EMB), jnp.float32),
    ],
    scratch_types=[
        pltpu.VMEM((_BPW,), jnp.int32),
        pltpu.VMEM((_BPW, EMB), jnp.float32),
        pltpu.SemaphoreType.DMA,
    ],
)
def _sc_gather(rest_t, menu_t, ridx, midx,
               rest_out, menu_out,
               idx_v, buf, sem):
    wid = lax.axis_index("s") * _NC + lax.axis_index("c")
    base = wid * _BPW

    for table, idx_hbm, out in ((rest_t, ridx, rest_out),
                                (menu_t, midx, menu_out)):
        pltpu.sync_copy(idx_hbm.at[pl.ds(base, _BPW)], idx_v)

        def body(g, _):
            v = idx_v[pl.ds(g * 16, 16)]
            for k in range(16):
                r = v[k]
                pltpu.async_copy(table.at[r], buf.at[g * 16 + k], sem)
            return 0
        lax.fori_loop(0, _BPW // 16, body, 0)

        pltpu.make_async_copy(out.at[pl.ds(base, _BPW), :], buf, sem).wait()
        pltpu.sync_copy(buf, out.at[pl.ds(base, _BPW), :])


def _dense_body(rest_ref, menu_ref, temp_ref, rb_ref, mb_ref, gb_ref,
                tW1_ref, tb1_ref, tW2_ref, tb2_ref, tW3_ref, tb3_ref,
                iW1r_ref, iW1m_ref, iW1t_ref, ib1_ref,
                iW2_ref, ib2_ref, iW3_ref, ib3_ref, out_ref):
    f32 = jnp.float32
    rest = rest_ref[...]
    menu = menu_ref[...]
    temp = temp_ref[...]
    mf = jnp.sum(rest * menu, axis=1, keepdims=True)
    h = jnp.maximum(
        jnp.dot(temp, tW1_ref[...], preferred_element_type=f32) + tb1_ref[...], 0.0)
    h = jnp.maximum(
        jnp.dot(h, tW2_ref[...], preferred_element_type=f32) + tb2_ref[...], 0.0)
    t_score = jnp.dot(h, tW3_ref[...], preferred_element_type=f32) + tb3_ref[...]
    g = (jnp.dot(rest, iW1r_ref[...], preferred_element_type=f32)
         + jnp.dot(menu, iW1m_ref[...], preferred_element_type=f32)
         + jnp.dot(temp, iW1t_ref[...], preferred_element_type=f32)
         + ib1_ref[...])
    g = jnp.maximum(g, 0.0)
    g = jnp.maximum(
        jnp.dot(g, iW2_ref[...], preferred_element_type=f32) + ib2_ref[...], 0.0)
    i_score = jnp.dot(g, iW3_ref[...], preferred_element_type=f32) + ib3_ref[...]
    pred = gb_ref[...] + rb_ref[...] + mb_ref[...] + mf + t_score + i_score
    out_ref[...] = jax.nn.sigmoid(pred[:, 0])


def kernel(restaurant_idx, menu_idx, temporal_features, rest_table, menu_table,
           rest_bias_table, menu_bias_table, global_bias, tW1, tb1, tW2, tb2,
           tW3, tb3, iW1, ib1, iW2, ib2, iW3, ib3):
    ridx = restaurant_idx.astype(jnp.int32)
    midx = menu_idx.astype(jnp.int32)
    rest_emb, menu_emb = _sc_gather(rest_table, menu_table, ridx, midx)
    rest_b = jnp.take(rest_bias_table, ridx, axis=0)
    menu_b = jnp.take(menu_bias_table, midx, axis=0)

    BS = 2048
    grid = (B // BS,)
    full = lambda shape: pl.BlockSpec(shape, lambda i: (0, 0))
    out = pl.pallas_call(
        _dense_body,
        grid=grid,
        in_specs=[
            pl.BlockSpec((BS, EMB), lambda i: (i, 0)),
            pl.BlockSpec((BS, EMB), lambda i: (i, 0)),
            pl.BlockSpec((BS, TDIM), lambda i: (i, 0)),
            pl.BlockSpec((BS, 1), lambda i: (i, 0)),
            pl.BlockSpec((BS, 1), lambda i: (i, 0)),
            full((1, 1)),
            full((TDIM, 32)), full((1, 32)),
            full((32, 16)), full((1, 16)),
            full((16, 1)), full((1, 1)),
            full((EMB, 128)), full((EMB, 128)), full((TDIM, 128)), full((1, 128)),
            full((128, 64)), full((1, 64)),
            full((64, 1)), full((1, 1)),
        ],
        out_specs=pl.BlockSpec((BS,), lambda i: (i,)),
        out_shape=jax.ShapeDtypeStruct((B,), jnp.float32),
    )(
        rest_emb, menu_emb, temporal_features, rest_b, menu_b,
        global_bias.reshape(1, 1),
        tW1, tb1.reshape(1, 32), tW2, tb2.reshape(1, 16),
        tW3, tb3.reshape(1, 1),
        iW1[:EMB], iW1[EMB:2 * EMB], iW1[2 * EMB:], ib1.reshape(1, 128),
        iW2, ib2.reshape(1, 64), iW3, ib3.reshape(1, 1),
    )
    return out
